# baseline (speedup 1.0000x reference)
import functools

import jax
import jax.numpy as jnp
from jax import lax
from jax.experimental import pallas as pl
from jax.experimental.pallas import tpu as pltpu
from jax.experimental.pallas import tpu_sc as plsc

B = 16384
EMB = 32
NC = 2
NS = 16
NW = NC * NS
B_PER_W = B // NW
CHUNK = 128
N_CHUNKS = B_PER_W // CHUNK

_sc_mesh = plsc.VectorSubcoreMesh(core_axis_name="c", subcore_axis_name="s")


@functools.partial(
    pl.kernel,
    out_type=[
        jax.ShapeDtypeStruct((B, EMB), jnp.float32),
        jax.ShapeDtypeStruct((B, EMB), jnp.float32),
    ],
    mesh=_sc_mesh,
    scratch_types=[
        pltpu.VMEM((N_CHUNKS, CHUNK), jnp.int32),
        pltpu.VMEM((N_CHUNKS, CHUNK), jnp.int32),
        pltpu.VMEM((B_PER_W, EMB), jnp.float32),
        pltpu.VMEM((B_PER_W, EMB), jnp.float32),
        pltpu.SemaphoreType.DMA,
        pltpu.SemaphoreType.DMA,
    ],
    compiler_params=pltpu.CompilerParams(use_tc_tiling_on_sc=False),
)
def _sc_gather(u_idx_hbm, i_idx_hbm, user_hbm, item_hbm,
               u_out, i_out,
               u_idx_v, i_idx_v, u_rows, i_rows, sem_u, sem_i):
    wid = lax.axis_index("s") * NC + lax.axis_index("c")
    base = wid * B_PER_W
    pltpu.sync_copy(u_idx_hbm.at[wid], u_idx_v)
    pltpu.sync_copy(i_idx_hbm.at[wid], i_idx_v)
    copies = []
    for j in range(N_CHUNKS):
        copies.append(pltpu.async_copy(
            user_hbm.at[u_idx_v.at[j]],
            u_rows.at[pl.ds(j * CHUNK, CHUNK)], sem_u))
        copies.append(pltpu.async_copy(
            item_hbm.at[i_idx_v.at[j]],
            i_rows.at[pl.ds(j * CHUNK, CHUNK)], sem_i))
    for c in copies:
        c.wait()
    pltpu.sync_copy(u_rows, u_out.at[pl.ds(base, B_PER_W)])
    pltpu.sync_copy(i_rows, i_out.at[pl.ds(base, B_PER_W)])


def _mlp_body(u_emb, i_emb, W1u, W1i, b1, W2, b2, W3, b3, W4, b4, out):
    h = jnp.maximum(
        u_emb[...] @ W1u[...] + i_emb[...] @ W1i[...] + b1[...], 0.0)
    h = jnp.maximum(h @ W2[...] + b2[...], 0.0)
    h = jnp.maximum(h @ W3[...] + b3[...], 0.0)
    out[...] = jax.nn.sigmoid(h @ W4[...] + b4[...])


def _mlp(u_emb, i_emb, W1u, W1i, b1, W2, b2, W3, b3, W4, b4):
    rows = 2048
    grid = B // rows
    full = lambda shape: pl.BlockSpec(shape, lambda n: (0, 0))
    return pl.pallas_call(
        _mlp_body,
        grid=(grid,),
        in_specs=[
            pl.BlockSpec((rows, EMB), lambda n: (n, 0)),
            pl.BlockSpec((rows, EMB), lambda n: (n, 0)),
            full((EMB, 32)), full((EMB, 32)), full((1, 32)),
            full((32, 16)), full((1, 16)),
            full((16, 8)), full((1, 8)),
            full((8, 1)), full((1, 1)),
        ],
        out_specs=pl.BlockSpec((rows, 1), lambda n: (n, 0)),
        out_shape=jax.ShapeDtypeStruct((B, 1), jnp.float32),
    )(u_emb, i_emb, W1u, W1i, b1, W2, b2, W3, b3, W4, b4)


@jax.jit
def kernel(u, i, user_table, item_table, W1, b1, W2, b2, W3, b3, W4, b4):
    u_idx = u.astype(jnp.int32).reshape(NW, N_CHUNKS, CHUNK)
    i_idx = i.astype(jnp.int32).reshape(NW, N_CHUNKS, CHUNK)
    u_emb, i_emb = _sc_gather(u_idx, i_idx, user_table, item_table)
    out = _mlp(
        u_emb, i_emb,
        W1[:EMB], W1[EMB:], b1.reshape(1, 32),
        W2, b2.reshape(1, 16),
        W3, b3.reshape(1, 8),
        W4, b4.reshape(1, 1),
    )
    return out.reshape(B)
